# initial kernel scaffold (unmeasured)
import jax
import jax.numpy as jnp
from jax import lax
from jax.experimental import pallas as pl
from jax.experimental.pallas import tpu as pltpu


def kernel(Q, K, V):
    b, s, h, d = Q.shape
    scale = d ** -0.5

    def body(q_ref, k_ref, v_ref, o_ref, kp_ref, vp_ref, send_sems, recv_sems):
        my_x = lax.axis_index("x")
        my_y = lax.axis_index("y")
        my_z = lax.axis_index("z")
        peer = (1 - my_x, my_y, my_z)

        barrier = pltpu.get_barrier_semaphore()
        pl.semaphore_signal(
            barrier, inc=1, device_id=peer,
            device_id_type=pl.DeviceIdType.MESH,
        )
        pl.semaphore_wait(barrier, 1)

        rdma_k = pltpu.make_async_remote_copy(
            src_ref=k_ref, dst_ref=kp_ref,
            send_sem=send_sems.at[0], recv_sem=recv_sems.at[0],
            device_id=peer, device_id_type=pl.DeviceIdType.MESH,
        )
        rdma_v = pltpu.make_async_remote_copy(
            src_ref=v_ref, dst_ref=vp_ref,
            send_sem=send_sems.at[1], recv_sem=recv_sems.at[1],
            device_id=peer, device_id_type=pl.DeviceIdType.MESH,
        )
        rdma_k.start()
        rdma_v.start()
        rdma_k.wait()
        rdma_v.wait()

        for bi in range(b):
            for hi in range(h):
                q2 = q_ref[bi, :, hi, :] * scale
                k0 = k_ref[bi, :, hi, :]
                k1 = kp_ref[bi, :, hi, :]
                s0 = lax.dot_general(
                    q2, k0, (((1,), (1,)), ((), ())),
                    preferred_element_type=jnp.float32,
                )
                s1 = lax.dot_general(
                    q2, k1, (((1,), (1,)), ((), ())),
                    preferred_element_type=jnp.float32,
                )
                st = jnp.concatenate([s0, s1], axis=-1)
                m = jnp.max(st, axis=-1, keepdims=True)
                p = jnp.exp(st - m)
                p = p / jnp.sum(p, axis=-1, keepdims=True)
                o = lax.dot_general(
                    p[:, :s], v_ref[bi, :, hi, :],
                    (((1,), (0,)), ((), ())),
                    preferred_element_type=jnp.float32,
                ) + lax.dot_general(
                    p[:, s:], vp_ref[bi, :, hi, :],
                    (((1,), (0,)), ((), ())),
                    preferred_element_type=jnp.float32,
                )
                o_ref[bi, :, hi, :] = o

    return pl.pallas_call(
        body,
        out_shape=jax.ShapeDtypeStruct((b, s, h, d), jnp.float32),
        in_specs=[
            pl.BlockSpec(memory_space=pltpu.VMEM),
            pl.BlockSpec(memory_space=pltpu.VMEM),
            pl.BlockSpec(memory_space=pltpu.VMEM),
        ],
        out_specs=pl.BlockSpec(memory_space=pltpu.VMEM),
        scratch_shapes=[
            pltpu.VMEM((b, s, h, d), jnp.float32),
            pltpu.VMEM((b, s, h, d), jnp.float32),
            pltpu.SemaphoreType.DMA((2,)),
            pltpu.SemaphoreType.DMA((2,)),
        ],
        compiler_params=pltpu.CompilerParams(collective_id=0),
    )(Q, K, V)


# baseline (device time: 154602 ns/iter reference)
import jax
import jax.numpy as jnp
from jax import lax
from jax.experimental import pallas as pl
from jax.experimental.pallas import tpu as pltpu


def kernel(Q, K, V):
    b, s, h, d = Q.shape
    scale = d ** -0.5
    hd = h * d

    def body(q_ref, k_ref, v_ref, o_ref, kp_ref, vp_ref, send_sems, recv_sems):
        my_x = lax.axis_index("x")
        my_y = lax.axis_index("y")
        my_z = lax.axis_index("z")
        peer = (1 - my_x, my_y, my_z)

        barrier = pltpu.get_barrier_semaphore()
        pl.semaphore_signal(
            barrier, inc=1, device_id=peer,
            device_id_type=pl.DeviceIdType.MESH,
        )
        pl.semaphore_wait(barrier, 1)

        rdma_k = pltpu.make_async_remote_copy(
            src_ref=k_ref, dst_ref=kp_ref,
            send_sem=send_sems.at[0], recv_sem=recv_sems.at[0],
            device_id=peer, device_id_type=pl.DeviceIdType.MESH,
        )
        rdma_v = pltpu.make_async_remote_copy(
            src_ref=v_ref, dst_ref=vp_ref,
            send_sem=send_sems.at[1], recv_sem=recv_sems.at[1],
            device_id=peer, device_id_type=pl.DeviceIdType.MESH,
        )
        rdma_k.start()
        rdma_v.start()
        rdma_k.wait()
        rdma_v.wait()

        for bi in range(b):
            for hi in range(h):
                sl = slice(hi * d, (hi + 1) * d)
                q2 = q_ref[bi, :, sl] * scale
                k0 = k_ref[bi, :, sl]
                k1 = kp_ref[bi, :, sl]
                s0 = lax.dot_general(
                    q2, k0, (((1,), (1,)), ((), ())),
                    preferred_element_type=jnp.float32,
                )
                s1 = lax.dot_general(
                    q2, k1, (((1,), (1,)), ((), ())),
                    preferred_element_type=jnp.float32,
                )
                st = jnp.concatenate([s0, s1], axis=-1)
                m = jnp.max(st, axis=-1, keepdims=True)
                p = jnp.exp(st - m)
                p = p / jnp.sum(p, axis=-1, keepdims=True)
                o = lax.dot_general(
                    p[:, :s], v_ref[bi, :, sl],
                    (((1,), (0,)), ((), ())),
                    preferred_element_type=jnp.float32,
                ) + lax.dot_general(
                    p[:, s:], vp_ref[bi, :, sl],
                    (((1,), (0,)), ((), ())),
                    preferred_element_type=jnp.float32,
                )
                o_ref[bi, :, sl] = o

    out = pl.pallas_call(
        body,
        out_shape=jax.ShapeDtypeStruct((b, s, hd), jnp.float32),
        in_specs=[
            pl.BlockSpec(memory_space=pltpu.VMEM),
            pl.BlockSpec(memory_space=pltpu.VMEM),
            pl.BlockSpec(memory_space=pltpu.VMEM),
        ],
        out_specs=pl.BlockSpec(memory_space=pltpu.VMEM),
        scratch_shapes=[
            pltpu.VMEM((b, s, hd), jnp.float32),
            pltpu.VMEM((b, s, hd), jnp.float32),
            pltpu.SemaphoreType.DMA((2,)),
            pltpu.SemaphoreType.DMA((2,)),
        ],
        compiler_params=pltpu.CompilerParams(
            collective_id=0,
            vmem_limit_bytes=100 * 1024 * 1024,
        ),
    )(Q.reshape(b, s, hd), K.reshape(b, s, hd), V.reshape(b, s, hd))
    return out.reshape(b, s, h, d)


# device time: 73220 ns/iter; 2.1115x vs baseline; 2.1115x over previous
import jax
import jax.numpy as jnp
from jax import lax
from jax.experimental import pallas as pl
from jax.experimental.pallas import tpu as pltpu


def kernel(Q, K, V):
    b, s, h, d = Q.shape
    scale = d ** -0.5
    hd = h * d
    hs = s // 2

    def body(q_ref, k_ref, v_ref, o_hbm, kp_ref, vp_ref, l_ref, oacc_ref,
             ostage_ref, send_sems, recv_sems, out_sems):
        my_x = lax.axis_index("x")
        my_y = lax.axis_index("y")
        my_z = lax.axis_index("z")
        peer = (1 - my_x, my_y, my_z)

        barrier = pltpu.get_barrier_semaphore()
        pl.semaphore_signal(
            barrier, inc=1, device_id=peer,
            device_id_type=pl.DeviceIdType.MESH,
        )
        pl.semaphore_wait(barrier, 1)

        def remote(src, dst, i):
            return pltpu.make_async_remote_copy(
                src_ref=src, dst_ref=dst,
                send_sem=send_sems.at[i], recv_sem=recv_sems.at[i],
                device_id=peer, device_id_type=pl.DeviceIdType.MESH,
            )

        rdmas = []
        for bi in range(b - 1):
            rdmas.append(remote(k_ref.at[bi], kp_ref.at[bi], 2 * bi))
            rdmas.append(remote(v_ref.at[bi], vp_ref.at[bi], 2 * bi + 1))
        nl = 2 * (b - 1)
        last = b - 1
        for ci in range(2):
            rows = pl.ds(ci * hs, hs)
            rdmas.append(remote(
                k_ref.at[last, rows], kp_ref.at[last, rows], nl + 2 * ci))
            rdmas.append(remote(
                v_ref.at[last, rows], vp_ref.at[last, rows], nl + 2 * ci + 1))
        for r in rdmas:
            r.start()

        for bi in range(b):
            for hi in range(h):
                sl = slice(hi * d, (hi + 1) * d)
                q2 = q_ref[bi, :, sl]
                s0 = lax.dot_general(
                    q2, k_ref[bi, :, sl], (((1,), (1,)), ((), ())),
                    preferred_element_type=jnp.float32,
                )
                p0 = jnp.exp(s0 * scale)
                l_ref[bi, :, hi:hi + 1] = jnp.sum(p0, axis=-1, keepdims=True)
                oacc_ref[bi, :, sl] = lax.dot_general(
                    p0.astype(jnp.bfloat16), v_ref[bi, :, sl],
                    (((1,), (0,)), ((), ())),
                    preferred_element_type=jnp.float32,
                )

        def head_block(bi, hi, krows):
            sl = slice(hi * d, (hi + 1) * d)
            q2 = q_ref[bi, :, sl]
            s1 = lax.dot_general(
                q2, kp_ref[bi, krows, sl], (((1,), (1,)), ((), ())),
                preferred_element_type=jnp.float32,
            )
            p1 = jnp.exp(s1 * scale)
            l1 = jnp.sum(p1, axis=-1, keepdims=True)
            o1 = lax.dot_general(
                p1.astype(jnp.bfloat16), vp_ref[bi, krows, sl],
                (((1,), (0,)), ((), ())),
                preferred_element_type=jnp.float32,
            )
            return o1, l1

        outcopies = []
        for bi in range(b - 1):
            rdmas[2 * bi].wait_recv()
            rdmas[2 * bi + 1].wait_recv()
            for hi in range(h):
                sl = slice(hi * d, (hi + 1) * d)
                o1, l1 = head_block(bi, hi, slice(None))
                ostage_ref[bi, :, sl] = (
                    (oacc_ref[bi, :, sl] + o1)
                    / (l_ref[bi, :, hi:hi + 1] + l1)
                ).astype(jnp.bfloat16)
            cp = pltpu.make_async_copy(
                ostage_ref.at[bi], o_hbm.at[bi], out_sems.at[bi])
            cp.start()
            outcopies.append(cp)

        rdmas[nl].wait_recv()
        rdmas[nl + 1].wait_recv()
        for hi in range(h):
            sl = slice(hi * d, (hi + 1) * d)
            o1, l1 = head_block(last, hi, pl.ds(0, hs))
            oacc_ref[last, :, sl] = oacc_ref[last, :, sl] + o1
            l_ref[last, :, hi:hi + 1] = l_ref[last, :, hi:hi + 1] + l1
        rdmas[nl + 2].wait_recv()
        rdmas[nl + 3].wait_recv()
        for hi in range(h):
            sl = slice(hi * d, (hi + 1) * d)
            o1, l1 = head_block(last, hi, pl.ds(hs, hs))
            ostage_ref[last, :, sl] = (
                (oacc_ref[last, :, sl] + o1)
                / (l_ref[last, :, hi:hi + 1] + l1)
            ).astype(jnp.bfloat16)
        cp = pltpu.make_async_copy(
            ostage_ref.at[last], o_hbm.at[last], out_sems.at[last])
        cp.start()
        outcopies.append(cp)

        for cp in outcopies:
            cp.wait()
        for r in rdmas:
            r.wait_send()

    out = pl.pallas_call(
        body,
        out_shape=jax.ShapeDtypeStruct((b, s, hd), jnp.bfloat16),
        in_specs=[
            pl.BlockSpec(memory_space=pltpu.VMEM),
            pl.BlockSpec(memory_space=pltpu.VMEM),
            pl.BlockSpec(memory_space=pltpu.VMEM),
        ],
        out_specs=pl.BlockSpec(memory_space=pltpu.MemorySpace.HBM),
        scratch_shapes=[
            pltpu.VMEM((b, s, hd), jnp.bfloat16),
            pltpu.VMEM((b, s, hd), jnp.bfloat16),
            pltpu.VMEM((b, s, h), jnp.float32),
            pltpu.VMEM((b, s, hd), jnp.float32),
            pltpu.VMEM((b, s, hd), jnp.bfloat16),
            pltpu.SemaphoreType.DMA((2 * b + 2,)),
            pltpu.SemaphoreType.DMA((2 * b + 2,)),
            pltpu.SemaphoreType.DMA((b,)),
        ],
        compiler_params=pltpu.CompilerParams(
            collective_id=0,
            vmem_limit_bytes=100 * 1024 * 1024,
        ),
    )(
        Q.astype(jnp.bfloat16).reshape(b, s, hd),
        K.astype(jnp.bfloat16).reshape(b, s, hd),
        V.astype(jnp.bfloat16).reshape(b, s, hd),
    )
    return out.reshape(b, s, h, d)
